# baseline (speedup 1.0000x reference)
import functools

import jax
import jax.numpy as jnp
from jax import lax
from jax.experimental import pallas as pl
from jax.experimental.pallas import tpu as pltpu
from jax.experimental.pallas import tpu_sc as plsc

NUM_TABLES = 26
VOCAB = 100000
EMBED_DIM = 32
BATCH = 4096
HOTNESS = 20

_L = 16
_NC = 2
_NS = 16
_NW = _NC * _NS
_BPW = BATCH // _NW
_ROWS_PER_CHUNK = 4
_IDX_PER_CHUNK = _ROWS_PER_CHUNK * HOTNESS
_NCHUNK = _BPW // _ROWS_PER_CHUNK
_IPT = _BPW * HOTNESS
_NBUF = 8


def _tree_sum(vals):
    while len(vals) > 1:
        nxt = [vals[i] + vals[i + 1] for i in range(0, len(vals) - 1, 2)]
        if len(vals) % 2:
            nxt.append(vals[-1])
        vals = nxt
    return vals[0]


def _body(tables_hbm, idx_hbm, out_hbm, *rest):
    idx_raw = rest[0]
    idx_bufs = rest[1:1 + _NBUF]
    stg_bufs = rest[1 + _NBUF:1 + 2 * _NBUF]
    acc = rest[1 + 2 * _NBUF]
    sems = rest[2 + 2 * _NBUF:2 + 3 * _NBUF]
    wid = lax.axis_index("s") * _NC + lax.axis_index("c")
    b0 = wid * _BPW

    def prep_and_fire(t_off, c, b):
        for i in range(_IDX_PER_CHUNK // _L):
            src = idx_raw[pl.ds(pl.multiple_of(c * _IDX_PER_CHUNK, 8) + i * _L,
                                _L)]
            idx_bufs[b][pl.ds(i * _L, _L)] = src + t_off
        pltpu.async_copy(tables_hbm.at[idx_bufs[b]], stg_bufs[b], sems[b])

    def wait_gather(b):
        pltpu.make_async_copy(tables_hbm.at[pl.ds(0, _IDX_PER_CHUNK)],
                              stg_bufs[b], sems[b]).wait()

    def accum(t, c, b):
        sref = stg_bufs[b]
        for p in range(_ROWS_PER_CHUNK):
            bl = c * _ROWS_PER_CHUNK + p
            for col in (0, _L):
                vals = [sref[p * HOTNESS + h, pl.ds(col, _L)]
                        for h in range(HOTNESS)]
                acc[bl, t, pl.ds(col, _L)] = _tree_sum(vals)

    def table_body(t, carry):
        t_off = t * VOCAB
        pltpu.sync_copy(
            idx_hbm.at[t, pl.ds(pl.multiple_of(b0 * HOTNESS, 8), _IPT)],
            idx_raw)
        for b in range(_NBUF):
            prep_and_fire(t_off, b, b)

        def chunk_body(cc, inner):
            for b in range(_NBUF):
                c = cc * _NBUF + b
                wait_gather(b)
                accum(t, c, b)

                @pl.when(c + _NBUF < _NCHUNK)
                def _():
                    prep_and_fire(t_off, c + _NBUF, b)
            return inner

        lax.fori_loop(0, _NCHUNK // _NBUF, chunk_body, 0)
        return carry

    lax.fori_loop(0, NUM_TABLES, table_body, 0)
    pltpu.sync_copy(acc, out_hbm.at[pl.ds(pl.multiple_of(b0, 8), _BPW)])


@functools.partial(jax.jit, static_argnums=())
def _run(tables_flat, idx):
    mesh = plsc.VectorSubcoreMesh(core_axis_name="c", subcore_axis_name="s")
    fn = pl.kernel(
        _body,
        out_type=jax.ShapeDtypeStruct((BATCH, NUM_TABLES, EMBED_DIM),
                                      jnp.float32),
        mesh=mesh,
        scratch_types=(
            [pltpu.VMEM((_IPT,), jnp.int32)]
            + [pltpu.VMEM((_IDX_PER_CHUNK,), jnp.int32)] * _NBUF
            + [pltpu.VMEM((_IDX_PER_CHUNK, EMBED_DIM), jnp.float32)] * _NBUF
            + [pltpu.VMEM((_BPW, NUM_TABLES, EMBED_DIM), jnp.float32)]
            + [pltpu.SemaphoreType.DMA] * _NBUF
        ),
        compiler_params=pltpu.CompilerParams(use_tc_tiling_on_sc=False),
    )
    return fn(tables_flat, idx)


def kernel(tables, inputs):
    tables_flat = tables.reshape(NUM_TABLES * VOCAB, EMBED_DIM)
    idx = inputs.astype(jnp.int32).reshape(NUM_TABLES, BATCH * HOTNESS)
    return _run(tables_flat, idx)


# speedup vs baseline: 1.1831x; 1.1831x over previous
import jax
import jax.numpy as jnp
from jax import lax
from jax.experimental import pallas as pl
from jax.experimental.pallas import tpu as pltpu
from jax.experimental.pallas import tpu_sc as plsc

NUM_TABLES = 26
VOCAB = 100000
EMBED_DIM = 32
BATCH = 4096
HOTNESS = 20

_L = 16
_NC = 2
_NS = 16
_NW = _NC * _NS
_BPW = BATCH // _NW
_IPT = _BPW * HOTNESS
_CH_ROWS = 640
_CH_POOL = _CH_ROWS // HOTNESS
_NCHUNK = _IPT // _CH_ROWS
_NG = NUM_TABLES * _NCHUNK
_ACC_GRP = 8


def _tree_sum(vals):
    while len(vals) > 1:
        nxt = [vals[i] + vals[i + 1] for i in range(0, len(vals) - 1, 2)]
        if len(vals) % 2:
            nxt.append(vals[-1])
        vals = nxt
    return vals[0]


def _body(tables_hbm, idx_hbm, out_hbm, ir0, ir1, io0, io1, st0, st1,
          ob0, ob1, gs0, gs1, is0, is1, os0, os1):
    wid = lax.axis_index("s") * _NC + lax.axis_index("c")
    b0 = wid * _BPW
    iraw = (ir0, ir1)
    iob = (io0, io1)
    stg = (st0, st1)
    outb = (ob0, ob1)
    gsem = (gs0, gs1)
    isem = (is0, is1)
    osem = (os0, os1)

    def fire_idx(t, p):
        pltpu.async_copy(
            idx_hbm.at[t, pl.ds(pl.multiple_of(b0 * HOTNESS, 8), _IPT)],
            iraw[p], isem[p])

    def wait_idx(p):
        pltpu.make_async_copy(idx_hbm.at[0, pl.ds(0, _IPT)], iraw[p],
                              isem[p]).wait()

    def prep_and_fire(g, b):
        t = g // _NCHUNK
        c = lax.rem(g, _NCHUNK)
        tp = lax.rem(t, 2)
        off = t * VOCAB
        for i in range(_CH_ROWS // _L):
            src0 = iraw[0][pl.ds(pl.multiple_of(c * _CH_ROWS, 8) + i * _L,
                                 _L)]
            src1 = iraw[1][pl.ds(pl.multiple_of(c * _CH_ROWS, 8) + i * _L,
                                 _L)]
            src = jnp.where(tp == 0, src0, src1)
            iob[b][pl.ds(i * _L, _L)] = src + off
        pltpu.async_copy(tables_hbm.at[iob[b]], stg[b], gsem[b])

    def wait_gather(b):
        pltpu.make_async_copy(tables_hbm.at[pl.ds(0, _CH_ROWS)], stg[b],
                              gsem[b]).wait()

    def accum(g, b):
        t = g // _NCHUNK
        c = lax.rem(g, _NCHUNK)
        tp = lax.rem(t, 2)
        sref = stg[b]

        def grp_body(gi, carry):
            for k in range(_ACC_GRP):
                p = gi * _ACC_GRP + k
                row = c * _CH_POOL + p
                for col in (0, _L):
                    vals = [sref[p * HOTNESS + h, pl.ds(col, _L)]
                            for h in range(HOTNESS)]
                    s = _tree_sum(vals)

                    @pl.when(tp == 0)
                    def _():
                        outb[0][row, pl.ds(col, _L)] = s

                    @pl.when(tp == 1)
                    def _():
                        outb[1][row, pl.ds(col, _L)] = s
            return carry

        lax.fori_loop(0, _CH_POOL // _ACC_GRP, grp_body, 0)

    fire_idx(0, 0)
    wait_idx(0)
    fire_idx(1, 1)

    off0 = 0
    for b in range(2):
        for i in range(_CH_ROWS // _L):
            iob[b][pl.ds(i * _L, _L)] = (
                iraw[0][pl.ds(b * _CH_ROWS + i * _L, _L)] + off0)
        pltpu.async_copy(tables_hbm.at[iob[b]], stg[b], gsem[b])

    def loop_body(gg, carry):
        for b in range(2):
            g = gg * 2 + b
            wait_gather(b)
            accum(g, b)
            t = g // _NCHUNK
            c = lax.rem(g, _NCHUNK)
            tp = lax.rem(t, 2)

            @pl.when(c == _NCHUNK - 1)
            def _():
                @pl.when(tp == 0)
                def _():
                    pltpu.async_copy(outb[0], out_hbm.at[pl.ds(b0, _BPW), t],
                                     osem[0])

                @pl.when(tp == 1)
                def _():
                    pltpu.async_copy(outb[1], out_hbm.at[pl.ds(b0, _BPW), t],
                                     osem[1])

            g2 = g + 2

            @pl.when(g2 < _NG)
            def _():
                t2 = g2 // _NCHUNK
                c2 = lax.rem(g2, _NCHUNK)
                tp2 = lax.rem(t2, 2)

                @pl.when(c2 == 0)
                def _():
                    @pl.when(tp2 == 0)
                    def _():
                        wait_idx(0)

                    @pl.when(tp2 == 1)
                    def _():
                        wait_idx(1)

                    @pl.when(t2 >= 2)
                    def _():
                        @pl.when(tp2 == 0)
                        def _():
                            pltpu.make_async_copy(
                                outb[0], out_hbm.at[pl.ds(b0, _BPW), 0],
                                osem[0]).wait()

                        @pl.when(tp2 == 1)
                        def _():
                            pltpu.make_async_copy(
                                outb[1], out_hbm.at[pl.ds(b0, _BPW), 0],
                                osem[1]).wait()

                    @pl.when(t2 + 1 < NUM_TABLES)
                    def _():
                        @pl.when(tp2 == 0)
                        def _():
                            fire_idx(t2 + 1, 1)

                        @pl.when(tp2 == 1)
                        def _():
                            fire_idx(t2 + 1, 0)

                prep_and_fire(g2, b)
        return carry

    lax.fori_loop(0, _NG // 2, loop_body, 0)

    pltpu.make_async_copy(outb[0], out_hbm.at[pl.ds(b0, _BPW), 0],
                          osem[0]).wait()
    pltpu.make_async_copy(outb[1], out_hbm.at[pl.ds(b0, _BPW), 0],
                          osem[1]).wait()


@jax.jit
def _run(tables_flat, idx):
    mesh = plsc.VectorSubcoreMesh(core_axis_name="c", subcore_axis_name="s")
    fn = pl.kernel(
        _body,
        out_type=jax.ShapeDtypeStruct((BATCH, NUM_TABLES, EMBED_DIM),
                                      jnp.float32),
        mesh=mesh,
        scratch_types=(
            [pltpu.VMEM((_IPT,), jnp.int32)] * 2
            + [pltpu.VMEM((_CH_ROWS,), jnp.int32)] * 2
            + [pltpu.VMEM((_CH_ROWS, EMBED_DIM), jnp.float32)] * 2
            + [pltpu.VMEM((_BPW, EMBED_DIM), jnp.float32)] * 2
            + [pltpu.SemaphoreType.DMA] * 6
        ),
        compiler_params=pltpu.CompilerParams(use_tc_tiling_on_sc=False),
    )
    return fn(tables_flat, idx)


def kernel(tables, inputs):
    tables_flat = tables.reshape(NUM_TABLES * VOCAB, EMBED_DIM)
    idx = inputs.astype(jnp.int32).reshape(NUM_TABLES, BATCH * HOTNESS)
    return _run(tables_flat, idx)


# speedup vs baseline: 1.1909x; 1.0066x over previous
import jax
import jax.numpy as jnp
from jax import lax
from jax.experimental import pallas as pl
from jax.experimental.pallas import tpu as pltpu
from jax.experimental.pallas import tpu_sc as plsc

NUM_TABLES = 26
VOCAB = 100000
EMBED_DIM = 32
BATCH = 4096
HOTNESS = 20

_L = 16
_NC = 2
_NS = 16
_NW = _NC * _NS
_BPW = BATCH // _NW
_IPT = _BPW * HOTNESS
_CH_ROWS = 1280
_CH_POOL = _CH_ROWS // HOTNESS
_NCHUNK = _IPT // _CH_ROWS
_NG = NUM_TABLES * _NCHUNK
_ACC_GRP = 8


def _tree_sum(vals):
    while len(vals) > 1:
        nxt = [vals[i] + vals[i + 1] for i in range(0, len(vals) - 1, 2)]
        if len(vals) % 2:
            nxt.append(vals[-1])
        vals = nxt
    return vals[0]


def _body(tables_hbm, idx_hbm, out_hbm, ir0, ir1, io0, io1, st0, st1,
          ob0, ob1, gs0, gs1, is0, is1, os0, os1):
    wid = lax.axis_index("s") * _NC + lax.axis_index("c")
    b0 = wid * _BPW
    iraw = (ir0, ir1)
    iob = (io0, io1)
    stg = (st0, st1)
    outb = (ob0, ob1)
    gsem = (gs0, gs1)
    isem = (is0, is1)
    osem = (os0, os1)

    def fire_idx(t, p):
        pltpu.async_copy(
            idx_hbm.at[t, pl.ds(pl.multiple_of(b0 * HOTNESS, 8), _IPT)],
            iraw[p], isem[p])

    def wait_idx(p):
        pltpu.make_async_copy(idx_hbm.at[0, pl.ds(0, _IPT)], iraw[p],
                              isem[p]).wait()

    def prep_and_fire(g, b):
        t = g // _NCHUNK
        c = lax.rem(g, _NCHUNK)
        tp = lax.rem(t, 2)
        off = t * VOCAB
        for i in range(_CH_ROWS // _L):
            src0 = iraw[0][pl.ds(pl.multiple_of(c * _CH_ROWS, 8) + i * _L,
                                 _L)]
            src1 = iraw[1][pl.ds(pl.multiple_of(c * _CH_ROWS, 8) + i * _L,
                                 _L)]
            src = jnp.where(tp == 0, src0, src1)
            iob[b][pl.ds(i * _L, _L)] = src + off
        pltpu.async_copy(tables_hbm.at[iob[b]], stg[b], gsem[b])

    def wait_gather(b):
        pltpu.make_async_copy(tables_hbm.at[pl.ds(0, _CH_ROWS)], stg[b],
                              gsem[b]).wait()

    def accum(g, b):
        t = g // _NCHUNK
        c = lax.rem(g, _NCHUNK)
        tp = lax.rem(t, 2)
        sref = stg[b]

        def grp_body(gi, carry):
            for k in range(_ACC_GRP):
                p = gi * _ACC_GRP + k
                row = c * _CH_POOL + p
                for col in (0, _L):
                    vals = [sref[p * HOTNESS + h, pl.ds(col, _L)]
                            for h in range(HOTNESS)]
                    s = _tree_sum(vals)

                    @pl.when(tp == 0)
                    def _():
                        outb[0][row, pl.ds(col, _L)] = s

                    @pl.when(tp == 1)
                    def _():
                        outb[1][row, pl.ds(col, _L)] = s
            return carry

        lax.fori_loop(0, _CH_POOL // _ACC_GRP, grp_body, 0)

    fire_idx(0, 0)
    wait_idx(0)
    fire_idx(1, 1)

    off0 = 0
    for b in range(2):
        for i in range(_CH_ROWS // _L):
            iob[b][pl.ds(i * _L, _L)] = (
                iraw[0][pl.ds(b * _CH_ROWS + i * _L, _L)] + off0)
        pltpu.async_copy(tables_hbm.at[iob[b]], stg[b], gsem[b])

    def loop_body(gg, carry):
        for b in range(2):
            g = gg * 2 + b
            wait_gather(b)
            accum(g, b)
            t = g // _NCHUNK
            c = lax.rem(g, _NCHUNK)
            tp = lax.rem(t, 2)

            @pl.when(c == _NCHUNK - 1)
            def _():
                @pl.when(tp == 0)
                def _():
                    pltpu.async_copy(outb[0], out_hbm.at[pl.ds(b0, _BPW), t],
                                     osem[0])

                @pl.when(tp == 1)
                def _():
                    pltpu.async_copy(outb[1], out_hbm.at[pl.ds(b0, _BPW), t],
                                     osem[1])

            g2 = g + 2

            @pl.when(g2 < _NG)
            def _():
                t2 = g2 // _NCHUNK
                c2 = lax.rem(g2, _NCHUNK)
                tp2 = lax.rem(t2, 2)

                @pl.when(c2 == 0)
                def _():
                    @pl.when(tp2 == 0)
                    def _():
                        wait_idx(0)

                    @pl.when(tp2 == 1)
                    def _():
                        wait_idx(1)

                    @pl.when(t2 >= 2)
                    def _():
                        @pl.when(tp2 == 0)
                        def _():
                            pltpu.make_async_copy(
                                outb[0], out_hbm.at[pl.ds(b0, _BPW), 0],
                                osem[0]).wait()

                        @pl.when(tp2 == 1)
                        def _():
                            pltpu.make_async_copy(
                                outb[1], out_hbm.at[pl.ds(b0, _BPW), 0],
                                osem[1]).wait()

                    @pl.when(t2 + 1 < NUM_TABLES)
                    def _():
                        @pl.when(tp2 == 0)
                        def _():
                            fire_idx(t2 + 1, 1)

                        @pl.when(tp2 == 1)
                        def _():
                            fire_idx(t2 + 1, 0)

                prep_and_fire(g2, b)
        return carry

    lax.fori_loop(0, _NG // 2, loop_body, 0)

    pltpu.make_async_copy(outb[0], out_hbm.at[pl.ds(b0, _BPW), 0],
                          osem[0]).wait()
    pltpu.make_async_copy(outb[1], out_hbm.at[pl.ds(b0, _BPW), 0],
                          osem[1]).wait()


@jax.jit
def _run(tables_flat, idx):
    mesh = plsc.VectorSubcoreMesh(core_axis_name="c", subcore_axis_name="s")
    fn = pl.kernel(
        _body,
        out_type=jax.ShapeDtypeStruct((BATCH, NUM_TABLES, EMBED_DIM),
                                      jnp.float32),
        mesh=mesh,
        scratch_types=(
            [pltpu.VMEM((_IPT,), jnp.int32)] * 2
            + [pltpu.VMEM((_CH_ROWS,), jnp.int32)] * 2
            + [pltpu.VMEM((_CH_ROWS, EMBED_DIM), jnp.float32)] * 2
            + [pltpu.VMEM((_BPW, EMBED_DIM), jnp.float32)] * 2
            + [pltpu.SemaphoreType.DMA] * 6
        ),
        compiler_params=pltpu.CompilerParams(use_tc_tiling_on_sc=False),
    )
    return fn(tables_flat, idx)


def kernel(tables, inputs):
    tables_flat = tables.reshape(NUM_TABLES * VOCAB, EMBED_DIM)
    idx = inputs.astype(jnp.int32).reshape(NUM_TABLES, BATCH * HOTNESS)
    return _run(tables_flat, idx)
